# baseline (speedup 1.0000x reference)
import jax
import jax.numpy as jnp
from jax.experimental import pallas as pl
from jax.experimental.pallas import tpu as pltpu


_B = 128
_SC = 40


def _copy_body(x_ref, o_ref):
    o_ref[...] = x_ref[...]


def kernel(tensor, mask):
    n, s, d = tensor.shape
    del mask
    return pl.pallas_call(
        _copy_body,
        grid=(n // _B, s // _SC),
        in_specs=[pl.BlockSpec((_B, _SC, d), lambda i, k: (i, k, 0))],
        out_specs=pl.BlockSpec((_B, _SC, d), lambda i, k: (i, k, 0)),
        out_shape=jax.ShapeDtypeStruct((n, s, d), tensor.dtype),
        compiler_params=pltpu.CompilerParams(
            dimension_semantics=("arbitrary", "arbitrary"),
        ),
    )(tensor)


# speedup vs baseline: 1.0304x; 1.0304x over previous
import jax
import jax.numpy as jnp
from jax.experimental import pallas as pl
from jax.experimental.pallas import tpu as pltpu


_BLOCK_ROWS = 128


def _fill_body(mask_ref, x_ref, o_ref):
    keep = 1.0 - mask_ref[...].astype(jnp.float32)
    o_ref[...] = x_ref[...] * keep[:, :, None]


def kernel(tensor, mask):
    n, s, d = tensor.shape
    b = _BLOCK_ROWS
    m8 = mask.view(jnp.uint8)
    return pl.pallas_call(
        _fill_body,
        grid=(n // b,),
        in_specs=[
            pl.BlockSpec((b, s), lambda i: (i, 0)),
            pl.BlockSpec((b, s, d), lambda i: (i, 0, 0)),
        ],
        out_specs=pl.BlockSpec((b, s, d), lambda i: (i, 0, 0)),
        out_shape=jax.ShapeDtypeStruct((n, s, d), tensor.dtype),
        compiler_params=pltpu.CompilerParams(
            dimension_semantics=("arbitrary",),
        ),
    )(m8, tensor)


# speedup vs baseline: 1.0307x; 1.0003x over previous
import jax
import jax.numpy as jnp
from jax.experimental import pallas as pl
from jax.experimental.pallas import tpu as pltpu


_BLOCK_ROWS = 128


def _fill_body(mask_ref, x_ref, o_ref):
    keep = 1.0 - mask_ref[...].astype(jnp.float32)
    o_ref[...] = x_ref[...] * keep[:, :, None]


def kernel(tensor, mask):
    n, s, d = tensor.shape
    b = _BLOCK_ROWS
    m8 = mask.view(jnp.uint8)
    return pl.pallas_call(
        _fill_body,
        grid=(n // b,),
        in_specs=[
            pl.BlockSpec((b, s), lambda i: (i, 0)),
            pl.BlockSpec((b, s, d), lambda i: (i, 0, 0)),
        ],
        out_specs=pl.BlockSpec((b, s, d), lambda i: (i, 0, 0)),
        out_shape=jax.ShapeDtypeStruct((n, s, d), tensor.dtype),
        compiler_params=pltpu.CompilerParams(
            dimension_semantics=("parallel",),
        ),
    )(m8, tensor)
